# Initial kernel scaffold; baseline (speedup 1.0000x reference)
import jax
import jax.numpy as jnp
from jax.experimental import pallas as pl


def kernel(query_rel, src_idx_0, rel_0, batch_idx_0, dst_idx_0, src_idx_1, rel_1, batch_idx_1, dst_idx_1, src_idx_2, rel_2, batch_idx_2, dst_idx_2, node_batch_idx, node_ent_idx, rela_embed, W1, W2, Wc, bc):
    raise NotImplementedError("write your pallas kernel here")


# speedup vs baseline: 1.3987x; 1.3987x over previous
import functools

import jax
import jax.numpy as jnp
from jax.experimental import pallas as pl

B = 1024
N_ENT = 10000
N_VOCAB = 402
N_NODES = 50000
E = 800000
DIM = 20
HID = 30

EDGE_BLK = 8000


def _edge_mlp_body(e_ref, r_ref, q_ref, w1_ref, w2_ref, out_ref):
    e = e_ref[...]
    r = r_ref[...]
    q = q_ref[...]
    att_in = jnp.concatenate([e, r, q], axis=1)
    h = jnp.maximum(att_in @ w1_ref[...].T, 0.0)
    score = jax.nn.sigmoid(h @ w2_ref[...].T)
    out_ref[...] = score * (e + r)


def _edge_mlp(e_src, r, q, w1, w2):
    n_blk = E // EDGE_BLK
    return pl.pallas_call(
        _edge_mlp_body,
        grid=(n_blk,),
        in_specs=[
            pl.BlockSpec((EDGE_BLK, DIM), lambda i: (i, 0)),
            pl.BlockSpec((EDGE_BLK, DIM), lambda i: (i, 0)),
            pl.BlockSpec((EDGE_BLK, DIM), lambda i: (i, 0)),
            pl.BlockSpec((HID, 3 * DIM), lambda i: (0, 0)),
            pl.BlockSpec((1, HID), lambda i: (0, 0)),
        ],
        out_specs=pl.BlockSpec((EDGE_BLK, DIM), lambda i: (i, 0)),
        out_shape=jax.ShapeDtypeStruct((E, DIM), jnp.float32),
    )(e_src, r, q, w1, w2)


def _layer(hidden, src, rel, bidx, dst, qe, rela_embed, w1, w2):
    e_src = jnp.take(hidden, src, axis=0)
    r = jnp.take(rela_embed, rel, axis=0)
    q = jnp.take(qe, bidx, axis=0)
    vals = _edge_mlp(e_src, r, q, w1, w2)
    return jax.ops.segment_sum(vals, dst, num_segments=N_NODES)


def kernel(query_rel, src_idx_0, rel_0, batch_idx_0, dst_idx_0,
           src_idx_1, rel_1, batch_idx_1, dst_idx_1,
           src_idx_2, rel_2, batch_idx_2, dst_idx_2,
           node_batch_idx, node_ent_idx, rela_embed, W1, W2, Wc, bc):
    qe = jnp.take(rela_embed, query_rel, axis=0)
    hidden = jnp.zeros((B, DIM), dtype=jnp.float32)
    hidden = _layer(hidden, src_idx_0, rel_0, batch_idx_0, dst_idx_0,
                    qe, rela_embed, W1, W2)
    hidden = _layer(hidden, src_idx_1, rel_1, batch_idx_1, dst_idx_1,
                    qe, rela_embed, W1, W2)
    hidden = _layer(hidden, src_idx_2, rel_2, batch_idx_2, dst_idx_2,
                    qe, rela_embed, W1, W2)
    result = (hidden @ Wc.T + bc).reshape(-1)
    score_all = jnp.zeros((B, N_ENT), dtype=jnp.float32).at[
        node_batch_idx, node_ent_idx].set(result)
    return score_all


# speedup vs baseline: 4.4808x; 3.2036x over previous
import functools

import jax
import jax.numpy as jnp
from jax import lax
from jax.experimental import pallas as pl
from jax.experimental.pallas import tpu as pltpu
from jax.experimental.pallas import tpu_sc as plsc

B = 1024
N_ENT = 10000
N_VOCAB = 402
N_NODES = 50000
E = 800000
DIM = 20
HID = 30
HTA_W = 48
HTB_W = 16
DB = DIM - 16

NC, NS, LANES = 2, 16, 16
NW = NC * NS
PER_W = E // NW
CHUNK = 200
CPAD = CHUNK + 8
N_GROUPS = CPAD // LANES
SPAN = (N_NODES + NS * 8 - 1) // (NS * 8) * 8


def _prep_body(qr_ref, re_ref, w1_ref, rb_ref, qc_ref, rea_ref, reb_ref):
    re = re_ref[...]
    w1 = w1_ref[...]
    w1b = w1[:, DIM:2 * DIM]
    w1c = w1[:, 2 * DIM:]
    qr = qr_ref[...]
    oh = (qr == lax.broadcasted_iota(jnp.int32, (B, N_VOCAB), 1)
          ).astype(jnp.float32)
    qe = oh @ re
    rb_ref[...] = re @ w1b.T
    qc = qe @ w1c.T
    qc_ref[...] = jnp.concatenate(
        [qc, jnp.zeros((B, 2), jnp.float32)], axis=1)
    rea_ref[...] = re[:, :16]
    reb_ref[...] = jnp.concatenate(
        [re[:, 16:], jnp.zeros((N_VOCAB, 16 - DB), jnp.float32)], axis=1)


def _prep(query_rel, rela_embed, W1):
    return pl.pallas_call(
        _prep_body,
        out_shape=(jax.ShapeDtypeStruct((N_VOCAB, HID), jnp.float32),
                   jax.ShapeDtypeStruct((B, 32), jnp.float32),
                   jax.ShapeDtypeStruct((N_VOCAB, 16), jnp.float32),
                   jax.ShapeDtypeStruct((N_VOCAB, 16), jnp.float32)),
    )(query_rel.astype(jnp.int32).reshape(B, 1), rela_embed, W1)


def _combine_body(pa_ref, pb_ref, w1_ref, hta_ref, htb_ref):
    h = jnp.concatenate([pa_ref[0] + pa_ref[1],
                         (pb_ref[0] + pb_ref[1])[:, :DB]], axis=1)
    w1a = w1_ref[...][:, :DIM]
    blk = h.shape[0]
    hta_ref[...] = jnp.concatenate(
        [h @ w1a.T, jnp.zeros((blk, 2), jnp.float32), h[:, :16]], axis=1)
    htb_ref[...] = jnp.concatenate(
        [h[:, 16:], jnp.zeros((blk, 16 - DB), jnp.float32)], axis=1)


def _combine(pa, pb, W1):
    blk = 5000
    return pl.pallas_call(
        _combine_body,
        grid=(N_NODES // blk,),
        in_specs=[pl.BlockSpec((2, blk, 16), lambda i: (0, i, 0)),
                  pl.BlockSpec((2, blk, 16), lambda i: (0, i, 0)),
                  pl.BlockSpec((HID, 3 * DIM), lambda i: (0, 0))],
        out_specs=[pl.BlockSpec((blk, HTA_W), lambda i: (i, 0)),
                   pl.BlockSpec((blk, HTB_W), lambda i: (i, 0))],
        out_shape=(jax.ShapeDtypeStruct((N_NODES, HTA_W), jnp.float32),
                   jax.ShapeDtypeStruct((N_NODES, HTB_W), jnp.float32)),
    )(pa, pb, W1)


def _result_body(pa_ref, pb_ref, wc_ref, bc_ref, out_ref):
    h = jnp.concatenate([pa_ref[0] + pa_ref[1],
                         (pb_ref[0] + pb_ref[1])[:, :DB]], axis=1)
    out_ref[...] = h @ wc_ref[...].T + bc_ref[0, 0]


def _result(pa, pb, Wc, bc):
    wcp = jnp.zeros((128, DIM), jnp.float32).at[0].set(Wc[0])
    blk = 5000
    out = pl.pallas_call(
        _result_body,
        grid=(N_NODES // blk,),
        in_specs=[pl.BlockSpec((2, blk, 16), lambda i: (0, i, 0)),
                  pl.BlockSpec((2, blk, 16), lambda i: (0, i, 0)),
                  pl.BlockSpec((128, DIM), lambda i: (0, 0)),
                  pl.BlockSpec((1, 1), lambda i: (0, 0))],
        out_specs=pl.BlockSpec((blk, 128), lambda i: (i, 0)),
        out_shape=jax.ShapeDtypeStruct((N_NODES, 128), jnp.float32),
    )(pa, pb, wcp, bc.reshape(1, 1))
    return out[:, 0]


def _zero_buf(buf, nwords):
    iota = lax.iota(jnp.int32, LANES)
    zf = jnp.zeros((LANES,), jnp.float32)
    w = buf.shape[1]

    def body(i, _):
        fid = i * LANES + iota
        plsc.store_scatter(buf, [fid // w, fid % w], zf)
        return 0
    lax.fori_loop(0, nwords // LANES, body, 0)


def _drain_spans(stage):
    spans = []
    k0 = 0
    while k0 < SPAN:
        spans.append((k0, min(stage, SPAN - k0)))
        k0 += stage
    return spans


def _sc_a_body(has_esrc, hta_hbm, src_hbm, rel_hbm, bidx_hbm, dst_hbm,
               rb_hbm, qc_hbm, rea_hbm, w2_hbm, out_hbm, sc_hbm,
               rb_v, rea_v, w2_v, src_v, rel_v, bidx_v, dst_v,
               rows_v, qcrows_v, vals_v, scores_v, qc_sh, accum, sem, sem2):
    c = lax.axis_index("c")
    s = lax.axis_index("s")
    wid = s * NC + c
    base = wid * PER_W
    iota = lax.iota(jnp.int32, LANES)
    zf = jnp.zeros((LANES,), jnp.float32)
    zi = jnp.zeros((LANES,), jnp.int32)

    pltpu.sync_copy(rb_hbm, rb_v)
    pltpu.sync_copy(rea_hbm, rea_v)
    pltpu.sync_copy(w2_hbm, w2_v)

    @pl.when(s == 0)
    def _stage_qc():
        pltpu.sync_copy(qc_hbm, qc_sh)

    _zero_buf(vals_v, CPAD * 16)
    row0 = pl.multiple_of(jnp.minimum(s * SPAN, N_NODES - SPAN), 8)
    spans = _drain_spans(CPAD)
    for k, n in spans:
        pltpu.sync_copy(vals_v.at[pl.ds(0, n)],
                        accum.at[pl.ds(row0 + k, n)])

    pad = pl.ds(CHUNK + 8 - LANES, LANES)
    rel_v[pad] = zi
    bidx_v[pad] = zi
    dst_v[pad] = zi
    if has_esrc:
        src_v[pad] = zi
    plsc.subcore_barrier()

    w2s = [plsc.load_gather(w2_v, [jnp.full((LANES,), j, jnp.int32)])
           for j in range(HID)]

    def group(g, masked):
        sl = pl.ds(g * LANES, LANES)
        rel16 = rel_v[sl]
        rowid = g * LANES + iota
        acc = zf
        for j in range(HID):
            jj = jnp.full((LANES,), j, jnp.int32)
            t = (plsc.load_gather(rb_v, [rel16, jj]) +
                 plsc.load_gather(qcrows_v, [rowid, jj]))
            if has_esrc:
                t = t + plsc.load_gather(rows_v, [rowid, jj])
            acc = acc + jnp.maximum(t, 0.0) * w2s[j]
        score = 1.0 / (1.0 + jnp.exp(-acc))
        if masked:
            score = jnp.where(rowid < CHUNK, score, 0.0)
        scores_v[sl] = score
        for d in range(16):
            dd = jnp.full((LANES,), d, jnp.int32)
            v = plsc.load_gather(rea_v, [rel16, dd])
            if has_esrc:
                v = v + plsc.load_gather(rows_v, [rowid, dd + 32])
            plsc.store_scatter(vals_v, [rowid, dd], score * v)

    def chunk_body(i, _):
        off = base + i * CHUNK
        cp = pl.ds(0, CHUNK)
        pltpu.sync_copy(rel_hbm.at[pl.ds(off, CHUNK)], rel_v.at[cp])
        pltpu.sync_copy(bidx_hbm.at[pl.ds(off, CHUNK)], bidx_v.at[cp])
        pltpu.sync_copy(dst_hbm.at[pl.ds(off, CHUNK)], dst_v.at[cp])
        qcp = pltpu.async_copy(qc_sh.at[bidx_v], qcrows_v, sem2)
        if has_esrc:
            pltpu.sync_copy(src_hbm.at[pl.ds(off, CHUNK)], src_v.at[cp])
            pltpu.async_copy(hta_hbm.at[src_v], rows_v, sem).wait()
        qcp.wait()

        def g_body(g, _):
            group(g, masked=False)
            return 0
        lax.fori_loop(0, N_GROUPS - 1, g_body, 0)
        group(N_GROUPS - 1, masked=True)

        pltpu.sync_copy(scores_v.at[cp], sc_hbm.at[pl.ds(off, CHUNK)])
        pltpu.sync_copy(vals_v, accum.at[dst_v], add=True)
        return 0

    lax.fori_loop(0, PER_W // CHUNK, chunk_body, 0)
    plsc.subcore_barrier()

    for k, n in spans:
        pltpu.sync_copy(accum.at[pl.ds(row0 + k, n)],
                        vals_v.at[pl.ds(0, n)])
        pltpu.sync_copy(vals_v.at[pl.ds(0, n)],
                        out_hbm.at[c, pl.ds(row0 + k, n)])


def _sc_a(has_esrc, hta, src, rel, bidx, dst, rb, qc, rea, w2pad):
    mesh = plsc.VectorSubcoreMesh(core_axis_name="c", subcore_axis_name="s")
    scratch = [
        pltpu.VMEM((N_VOCAB, HID), jnp.float32),
        pltpu.VMEM((N_VOCAB, 16), jnp.float32),
        pltpu.VMEM((2 * LANES,), jnp.float32),
        pltpu.VMEM((CPAD,), jnp.int32),
        pltpu.VMEM((CPAD,), jnp.int32),
        pltpu.VMEM((CPAD,), jnp.int32),
        pltpu.VMEM((CPAD,), jnp.int32),
        pltpu.VMEM((CPAD if has_esrc else 8, HTA_W), jnp.float32),
        pltpu.VMEM((CPAD, 32), jnp.float32),
        pltpu.VMEM((CPAD, 16), jnp.float32),
        pltpu.VMEM((CPAD,), jnp.float32),
        pltpu.VMEM_SHARED((B, 32), jnp.float32),
        pltpu.VMEM_SHARED((N_NODES, 16), jnp.float32),
        pltpu.SemaphoreType.DMA,
        pltpu.SemaphoreType.DMA,
    ]
    fn = pl.kernel(
        functools.partial(_sc_a_body, has_esrc),
        mesh=mesh,
        compiler_params=pltpu.CompilerParams(
            use_tc_tiling_on_sc=False, needs_layout_passes=False),
        out_type=(jax.ShapeDtypeStruct((NC, N_NODES, 16), jnp.float32),
                  jax.ShapeDtypeStruct((E,), jnp.float32)),
        scratch_types=scratch,
    )
    return fn(hta, src, rel, bidx, dst, rb, qc, rea, w2pad)


def _sc_b_body(has_esrc, htb_hbm, src_hbm, rel_hbm, dst_hbm, sc_hbm,
               reb_hbm, out_hbm,
               reb_v, src_v, rel_v, dst_v, rows_v, vals_v, scores_v,
               accum, sem):
    c = lax.axis_index("c")
    s = lax.axis_index("s")
    wid = s * NC + c
    base = wid * PER_W
    iota = lax.iota(jnp.int32, LANES)
    zi = jnp.zeros((LANES,), jnp.int32)

    pltpu.sync_copy(reb_hbm, reb_v)

    _zero_buf(vals_v, CPAD * 16)
    row0 = pl.multiple_of(jnp.minimum(s * SPAN, N_NODES - SPAN), 8)
    spans = _drain_spans(CPAD)
    for k, n in spans:
        pltpu.sync_copy(vals_v.at[pl.ds(0, n)],
                        accum.at[pl.ds(row0 + k, n)])

    pad = pl.ds(CHUNK + 8 - LANES, LANES)
    rel_v[pad] = zi
    dst_v[pad] = zi
    if has_esrc:
        src_v[pad] = zi
    plsc.subcore_barrier()

    def group(g, masked):
        sl = pl.ds(g * LANES, LANES)
        rel16 = rel_v[sl]
        rowid = g * LANES + iota
        score = scores_v[sl]
        if masked:
            score = jnp.where(rowid < CHUNK, score, 0.0)
        for d in range(16):
            dd = jnp.full((LANES,), d, jnp.int32)
            v = plsc.load_gather(reb_v, [rel16, dd])
            if has_esrc:
                v = v + plsc.load_gather(rows_v, [rowid, dd])
            plsc.store_scatter(vals_v, [rowid, dd], score * v)

    def chunk_body(i, _):
        off = base + i * CHUNK
        cp = pl.ds(0, CHUNK)
        pltpu.sync_copy(rel_hbm.at[pl.ds(off, CHUNK)], rel_v.at[cp])
        pltpu.sync_copy(dst_hbm.at[pl.ds(off, CHUNK)], dst_v.at[cp])
        pltpu.sync_copy(sc_hbm.at[pl.ds(off, CHUNK)], scores_v.at[cp])
        if has_esrc:
            pltpu.sync_copy(src_hbm.at[pl.ds(off, CHUNK)], src_v.at[cp])
            pltpu.async_copy(htb_hbm.at[src_v], rows_v, sem).wait()

        def g_body(g, _):
            group(g, masked=False)
            return 0
        lax.fori_loop(0, N_GROUPS - 1, g_body, 0)
        group(N_GROUPS - 1, masked=True)

        pltpu.sync_copy(vals_v, accum.at[dst_v], add=True)
        return 0

    lax.fori_loop(0, PER_W // CHUNK, chunk_body, 0)
    plsc.subcore_barrier()

    for k, n in spans:
        pltpu.sync_copy(accum.at[pl.ds(row0 + k, n)],
                        vals_v.at[pl.ds(0, n)])
        pltpu.sync_copy(vals_v.at[pl.ds(0, n)],
                        out_hbm.at[c, pl.ds(row0 + k, n)])


def _sc_b(has_esrc, htb, src, rel, dst, scores, reb):
    mesh = plsc.VectorSubcoreMesh(core_axis_name="c", subcore_axis_name="s")
    scratch = [
        pltpu.VMEM((N_VOCAB, 16), jnp.float32),
        pltpu.VMEM((CPAD,), jnp.int32),
        pltpu.VMEM((CPAD,), jnp.int32),
        pltpu.VMEM((CPAD,), jnp.int32),
        pltpu.VMEM((CPAD if has_esrc else 8, HTB_W), jnp.float32),
        pltpu.VMEM((CPAD, 16), jnp.float32),
        pltpu.VMEM((CPAD,), jnp.float32),
        pltpu.VMEM_SHARED((N_NODES, 16), jnp.float32),
        pltpu.SemaphoreType.DMA,
    ]
    fn = pl.kernel(
        functools.partial(_sc_b_body, has_esrc),
        mesh=mesh,
        compiler_params=pltpu.CompilerParams(
            use_tc_tiling_on_sc=False, needs_layout_passes=False),
        out_type=jax.ShapeDtypeStruct((NC, N_NODES, 16), jnp.float32),
        scratch_types=scratch,
    )
    return fn(htb, src, rel, dst, scores, reb)


def _sc_layer(has_esrc, hta, htb, src, rel, bidx, dst, rb, qc, rea, reb,
              w2pad):
    pa, scores = _sc_a(has_esrc, hta, src, rel, bidx, dst, rb, qc, rea,
                       w2pad)
    pb = _sc_b(has_esrc, htb, src, rel, dst, scores, reb)
    return pa, pb


def kernel(query_rel, src_idx_0, rel_0, batch_idx_0, dst_idx_0,
           src_idx_1, rel_1, batch_idx_1, dst_idx_1,
           src_idx_2, rel_2, batch_idx_2, dst_idx_2,
           node_batch_idx, node_ent_idx, rela_embed, W1, W2, Wc, bc):
    rb, qc, rea, reb = _prep(query_rel, rela_embed, W1)
    w2pad = jnp.zeros((2 * LANES,), jnp.float32).at[:HID].set(W2[0])
    dummy_a = jnp.zeros((8, HTA_W), jnp.float32)
    dummy_b = jnp.zeros((8, HTB_W), jnp.float32)

    pa, pb = _sc_layer(False, dummy_a, dummy_b, src_idx_0, rel_0,
                       batch_idx_0, dst_idx_0, rb, qc, rea, reb, w2pad)
    hta, htb = _combine(pa, pb, W1)
    pa, pb = _sc_layer(True, hta, htb, src_idx_1, rel_1,
                       batch_idx_1, dst_idx_1, rb, qc, rea, reb, w2pad)
    hta, htb = _combine(pa, pb, W1)
    pa, pb = _sc_layer(True, hta, htb, src_idx_2, rel_2,
                       batch_idx_2, dst_idx_2, rb, qc, rea, reb, w2pad)
    result = _result(pa, pb, Wc, bc)

    score_all = jnp.zeros((B, N_ENT), dtype=jnp.float32).at[
        node_batch_idx, node_ent_idx].set(result)
    return score_all
